# Initial kernel scaffold; baseline (speedup 1.0000x reference)
import jax
import jax.numpy as jnp
from jax.experimental import pallas as pl


def kernel(node_feats, neighbor_idx, W_ih_r, W_hh_r, b_ih_r, b_hh_r, W_ih_l, W_hh_l, b_ih_l, b_hh_l, W_lin, b_lin, bias):
    raise NotImplementedError("write your pallas kernel here")


# speedup vs baseline: 7.8914x; 7.8914x over previous
import functools

import jax
import jax.numpy as jnp
from jax import lax
from jax.experimental import pallas as pl
from jax.experimental.pallas import tpu as pltpu
from jax.experimental.pallas import tpu_sc as plsc

N = 10000
DEG = 16
F = 128
G4 = 4 * F
NP = 10240
TOT = DEG * NP

NW = 32
ROWS_PER_W = TOT // NW
CH = 128
NCH = ROWS_PER_W // CH


def _sc_gather_body(idx_hbm, table_hbm, out_hbm, idx_v, rows_v, sem):
    wid = lax.axis_index("s") * 2 + lax.axis_index("c")
    pltpu.sync_copy(idx_hbm.at[wid], idx_v)
    base = wid * ROWS_PER_W

    def chunk(ci, carry):
        pltpu.async_copy(table_hbm.at[idx_v.at[ci]], rows_v, sem).wait()
        pltpu.sync_copy(rows_v, out_hbm.at[pl.ds(base + ci * CH, CH)])
        return carry

    lax.fori_loop(0, NCH, chunk, 0)


def _sc_gather(idx3, table):
    mesh = plsc.VectorSubcoreMesh(core_axis_name="c", subcore_axis_name="s")
    kfn = functools.partial(
        pl.kernel,
        mesh=mesh,
        out_type=jax.ShapeDtypeStruct((TOT, F), jnp.float32),
        scratch_types=[
            pltpu.VMEM((NCH, CH), jnp.int32),
            pltpu.VMEM((CH, F), jnp.float32),
            pltpu.SemaphoreType.DMA,
        ],
    )(_sc_gather_body)
    return kfn(idx3, table)


BN = 512
NBLK = NP // BN


def _reducer_body(mb_ref, wih_ref, whh_ref, br_ref, wil_ref, bl_ref, xp_ref):
    h = jnp.zeros((BN, F), jnp.float32)
    c = jnp.zeros((BN, F), jnp.float32)
    wih = wih_ref[...]
    whh = whh_ref[...]
    br = br_ref[...]
    for t in range(DEG):
        x = mb_ref[t]
        g = (jnp.dot(x, wih, preferred_element_type=jnp.float32)
             + jnp.dot(h, whh, preferred_element_type=jnp.float32) + br)
        i = jax.nn.sigmoid(g[:, 0:F])
        f = jax.nn.sigmoid(g[:, F:2 * F])
        gg = jnp.tanh(g[:, 2 * F:3 * F])
        o = jax.nn.sigmoid(g[:, 3 * F:4 * F])
        c = f * c + i * gg
        h = o * jnp.tanh(c)
    xp_ref[...] = (jnp.dot(h, wil_ref[...], preferred_element_type=jnp.float32)
                   + bl_ref[...])


def _reducer(mb3, wihrT, whhrT, br, wihlT, bl):
    return pl.pallas_call(
        _reducer_body,
        grid=(NBLK,),
        in_specs=[
            pl.BlockSpec((DEG, BN, F), lambda b: (0, b, 0)),
            pl.BlockSpec((F, G4), lambda b: (0, 0)),
            pl.BlockSpec((F, G4), lambda b: (0, 0)),
            pl.BlockSpec((1, G4), lambda b: (0, 0)),
            pl.BlockSpec((F, G4), lambda b: (0, 0)),
            pl.BlockSpec((1, G4), lambda b: (0, 0)),
        ],
        out_specs=pl.BlockSpec((BN, G4), lambda b: (b, 0)),
        out_shape=jax.ShapeDtypeStruct((NP, G4), jnp.float32),
    )(mb3, wihrT, whhrT, br, wihlT, bl)


CHUNK = 1280
NSEQ = NP // CHUNK


def _seq_body(xp_ref, whh_ref, wlin_ref, blin_ref, out_ref, h_s, c_s, hs_s):
    b = pl.program_id(0)

    @pl.when(b == 0)
    def _():
        h_s[...] = jnp.zeros((1, F), jnp.float32)
        c_s[...] = jnp.zeros((1, F), jnp.float32)

    steps = jnp.minimum(CHUNK, N - b * CHUNK)
    whh = whh_ref[...]

    def step(i, carry):
        h, c = carry
        g = xp_ref[pl.ds(i, 1), :] + jnp.dot(h, whh,
                                             preferred_element_type=jnp.float32)
        ii = jax.nn.sigmoid(g[:, 0:F])
        f = jax.nn.sigmoid(g[:, F:2 * F])
        gg = jnp.tanh(g[:, 2 * F:3 * F])
        o = jax.nn.sigmoid(g[:, 3 * F:4 * F])
        c = f * c + ii * gg
        h = o * jnp.tanh(c)
        hs_s[pl.ds(i, 1), :] = h
        return (h, c)

    h, c = lax.fori_loop(0, steps, step, (h_s[...], c_s[...]))
    h_s[...] = h
    c_s[...] = c
    out_ref[...] = (jnp.dot(hs_s[...], wlin_ref[...],
                            preferred_element_type=jnp.float32) + blin_ref[...])


def _seq(xp, whhlT, wlinT, blin):
    return pl.pallas_call(
        _seq_body,
        grid=(NSEQ,),
        in_specs=[
            pl.BlockSpec((CHUNK, G4), lambda b: (b, 0)),
            pl.BlockSpec((F, G4), lambda b: (0, 0)),
            pl.BlockSpec((F, F), lambda b: (0, 0)),
            pl.BlockSpec((1, F), lambda b: (0, 0)),
        ],
        out_specs=pl.BlockSpec((CHUNK, F), lambda b: (b, 0)),
        out_shape=jax.ShapeDtypeStruct((NP, F), jnp.float32),
        scratch_shapes=[
            pltpu.VMEM((1, F), jnp.float32),
            pltpu.VMEM((1, F), jnp.float32),
            pltpu.VMEM((CHUNK, F), jnp.float32),
        ],
    )(xp, whhlT, wlinT, blin)


def kernel(node_feats, neighbor_idx, W_ih_r, W_hh_r, b_ih_r, b_hh_r,
           W_ih_l, W_hh_l, b_ih_l, b_hh_l, W_lin, b_lin, bias):
    idx = jnp.transpose(neighbor_idx)
    idx = jnp.pad(idx, ((0, 0), (0, NP - N)))
    idx3 = idx.reshape(NW, NCH, CH)

    mb = _sc_gather(idx3, node_feats)
    mb3 = mb.reshape(DEG, NP, F)

    wihrT = jnp.transpose(W_ih_r)
    whhrT = jnp.transpose(W_hh_r)
    br = (b_ih_r + b_hh_r)[None, :]
    wihlT = jnp.transpose(W_ih_l)
    bl = (b_ih_l + b_hh_l)[None, :]
    whhlT = jnp.transpose(W_hh_l)
    wlinT = jnp.transpose(W_lin)
    blin = (b_lin + bias)[None, :]

    xp = _reducer(mb3, wihrT, whhrT, br, wihlT, bl)
    out = _seq(xp, whhlT, wlinT, blin)
    return out[:N]
